# baseline (speedup 1.0000x reference)
import jax
import jax.numpy as jnp
from jax.experimental import pallas as pl
from jax.experimental.pallas import tpu as pltpu

_N = 100000
_F = 128
_B = 1024
_NJ = 20
_CHUNK = 12800
_GRID = 8


def _nmf_kernel(i_ref, js_ref, w_ref, e_ref, out_ref, p_acc):
    n = pl.program_id(0)
    base = n * _CHUNK

    row_idx = jax.lax.broadcasted_iota(jnp.int32, (_CHUNK, 1), 0) + base
    w_blk = jnp.where(row_idx < _N, w_ref[...], 0.0)
    col_idx = jax.lax.broadcasted_iota(jnp.int32, (1, _CHUNK), 1) + base
    e_blk = jnp.where(col_idx < _N, e_ref[...], 0.0)

    p_part = jnp.dot(e_blk.astype(jnp.bfloat16), w_blk.astype(jnp.bfloat16),
                     preferred_element_type=jnp.float32)

    @pl.when(n == 0)
    def _init():
        p_acc[...] = p_part

    @pl.when(n != 0)
    def _accum():
        p_acc[...] += p_part

    @pl.when(n == _GRID - 1)
    def _finish():
        js_row = js_ref[...]
        feat = jax.lax.broadcasted_iota(jnp.int32, (_F, _NJ), 0)
        m_col = jnp.sum((js_row == feat).astype(jnp.float32), axis=1,
                        keepdims=True)
        d_col = jnp.dot(p_acc[...], m_col,
                        preferred_element_type=jnp.float32)
        lane = jax.lax.broadcasted_iota(jnp.int32, (_B, _F), 1)
        onehot_i = (i_ref[...] == lane).astype(jnp.float32)
        out_ref[...] = jnp.dot(onehot_i, d_col,
                               preferred_element_type=jnp.float32)


def kernel(i, js, W, E):
    i2 = i.astype(jnp.int32).reshape(_B, 1)
    js2 = js.astype(jnp.int32).reshape(1, _NJ)
    out = pl.pallas_call(
        _nmf_kernel,
        grid=(_GRID,),
        in_specs=[
            pl.BlockSpec((_B, 1), lambda n: (0, 0)),
            pl.BlockSpec((1, _NJ), lambda n: (0, 0)),
            pl.BlockSpec((_CHUNK, _F), lambda n: (n, 0)),
            pl.BlockSpec((_F, _CHUNK), lambda n: (0, n)),
        ],
        out_specs=pl.BlockSpec((_B, 1), lambda n: (0, 0)),
        out_shape=jax.ShapeDtypeStruct((_B, 1), jnp.float32),
        scratch_shapes=[pltpu.VMEM((_F, _F), jnp.float32)],
        compiler_params=pltpu.CompilerParams(
            dimension_semantics=("arbitrary",),
        ),
    )(i2, js2, W, E)
    return out.reshape(_B)


# speedup vs baseline: 1.0157x; 1.0157x over previous
import jax
import jax.numpy as jnp
from jax.experimental import pallas as pl
from jax.experimental.pallas import tpu as pltpu

_N = 100000
_F = 128
_B = 1024
_NJ = 20
_CHUNK = 6400
_GRID = 16


def _nmf_kernel(i_ref, js_ref, w_ref, e_ref, out_ref, p_acc):
    n = pl.program_id(0)
    base = n * _CHUNK

    row_idx = jax.lax.broadcasted_iota(jnp.int32, (_CHUNK, 1), 0) + base
    w_blk = jnp.where(row_idx < _N, w_ref[...], 0.0)
    col_idx = jax.lax.broadcasted_iota(jnp.int32, (1, _CHUNK), 1) + base
    e_blk = jnp.where(col_idx < _N, e_ref[...], 0.0)

    p_part = jnp.dot(e_blk.astype(jnp.bfloat16), w_blk.astype(jnp.bfloat16),
                     preferred_element_type=jnp.float32)

    @pl.when(n == 0)
    def _init():
        p_acc[...] = p_part

    @pl.when(n != 0)
    def _accum():
        p_acc[...] += p_part

    @pl.when(n == _GRID - 1)
    def _finish():
        js_row = js_ref[...]
        feat = jax.lax.broadcasted_iota(jnp.int32, (_F, _NJ), 0)
        m_col = jnp.sum((js_row == feat).astype(jnp.float32), axis=1,
                        keepdims=True)
        d_col = jnp.dot(p_acc[...], m_col,
                        preferred_element_type=jnp.float32)
        lane = jax.lax.broadcasted_iota(jnp.int32, (_B, _F), 1)
        onehot_i = (i_ref[...] == lane).astype(jnp.float32)
        out_ref[...] = jnp.dot(onehot_i, d_col,
                               preferred_element_type=jnp.float32)


def kernel(i, js, W, E):
    i2 = i.astype(jnp.int32).reshape(_B, 1)
    js2 = js.astype(jnp.int32).reshape(1, _NJ)
    out = pl.pallas_call(
        _nmf_kernel,
        grid=(_GRID,),
        in_specs=[
            pl.BlockSpec((_B, 1), lambda n: (0, 0)),
            pl.BlockSpec((1, _NJ), lambda n: (0, 0)),
            pl.BlockSpec((_CHUNK, _F), lambda n: (n, 0)),
            pl.BlockSpec((_F, _CHUNK), lambda n: (0, n)),
        ],
        out_specs=pl.BlockSpec((_B, 1), lambda n: (0, 0)),
        out_shape=jax.ShapeDtypeStruct((_B, 1), jnp.float32),
        scratch_shapes=[pltpu.VMEM((_F, _F), jnp.float32)],
        compiler_params=pltpu.CompilerParams(
            dimension_semantics=("arbitrary",),
        ),
    )(i2, js2, W, E)
    return out.reshape(_B)
